# baseline (speedup 1.0000x reference)
import jax
import jax.numpy as jnp
from jax import lax
from jax.experimental import pallas as pl
from jax.experimental.pallas import tpu as pltpu
from jax.experimental.pallas import tpu_sc as plsc

B = 16384
D = 64
MARGIN = 2.0
L = 16
NC, NS = 2, 16
NW = NC * NS
BPW = B // NW
C = 128
NCHUNK = BPW // C
S = 64
NSUB = BPW // S
PD = 2 * D


def _body(heads, relations, tails, h_hat, t_hat, ent, rel, out_hbm,
          idx_h, idx_r, idx_t, idx_hh, idx_th,
          half_h, half_r, half_t, half_hh, half_th,
          rows, out_v, sem):
    wid = lax.axis_index("s") * NC + lax.axis_index("c")
    base = wid * BPW

    def stage(ci, carry):
        off = base + ci * C
        dst = pl.ds(ci * C, C)
        icps = [
            pltpu.async_copy(heads.at[pl.ds(off, C)], idx_h.at[dst], sem),
            pltpu.async_copy(relations.at[pl.ds(off, C)], idx_r.at[dst], sem),
            pltpu.async_copy(tails.at[pl.ds(off, C)], idx_t.at[dst], sem),
            pltpu.async_copy(h_hat.at[pl.ds(off, C)], idx_hh.at[dst], sem),
            pltpu.async_copy(t_hat.at[pl.ds(off, C)], idx_th.at[dst], sem),
        ]
        for cp in icps:
            cp.wait()
        return carry

    lax.fori_loop(0, NCHUNK, stage, 0)

    def halve(i, carry):
        sl = pl.ds(i * L, L)
        half_h[sl] = idx_h[sl] >> 1
        half_r[sl] = idx_r[sl] >> 1
        half_t[sl] = idx_t[sl] >> 1
        half_hh[sl] = idx_hh[sl] >> 1
        half_th[sl] = idx_th[sl] >> 1
        return carry

    lax.fori_loop(0, BPW // L, halve, 0)

    def fire(si, buf):
        sl = pl.ds(si * S, S)
        return [
            pltpu.async_copy(ent.at[half_h.at[sl]], rows.at[buf, 0], sem),
            pltpu.async_copy(rel.at[half_r.at[sl]], rows.at[buf, 1], sem),
            pltpu.async_copy(ent.at[half_t.at[sl]], rows.at[buf, 2], sem),
            pltpu.async_copy(ent.at[half_hh.at[sl]], rows.at[buf, 3], sem),
            pltpu.async_copy(ent.at[half_th.at[sl]], rows.at[buf, 4], sem),
        ]

    def compute(si, buf):
        def group(g, gcarry):
            sl = pl.ds(si * S + g * L, L)
            row_ids = g * L + lax.iota(jnp.int32, L)
            bh = (idx_h[sl] & 1) * D
            br = (idx_r[sl] & 1) * D
            bt = (idx_t[sl] & 1) * D
            bhh = (idx_hh[sl] & 1) * D
            bth = (idx_th[sl] & 1) * D
            zero = jnp.zeros((L,), jnp.float32)
            buf_h = rows.at[buf, 0]
            buf_r = rows.at[buf, 1]
            buf_t = rows.at[buf, 2]
            buf_hh = rows.at[buf, 3]
            buf_th = rows.at[buf, 4]

            def cols(j, dcarry):
                d1a, d2a, d1b, d2b = dcarry
                rv = plsc.load_gather(buf_r, [row_ids, br + j])
                hv = plsc.load_gather(buf_h, [row_ids, bh + j])
                tv = plsc.load_gather(buf_t, [row_ids, bt + j])
                hhv = plsc.load_gather(buf_hh, [row_ids, bhh + j])
                thv = plsc.load_gather(buf_th, [row_ids, bth + j])
                d1a = d1a + jnp.abs(hv + rv - tv)
                d2a = d2a + jnp.abs(hhv + rv - thv)
                rv2 = plsc.load_gather(buf_r, [row_ids, br + (j + 1)])
                hv2 = plsc.load_gather(buf_h, [row_ids, bh + (j + 1)])
                tv2 = plsc.load_gather(buf_t, [row_ids, bt + (j + 1)])
                hhv2 = plsc.load_gather(buf_hh, [row_ids, bhh + (j + 1)])
                thv2 = plsc.load_gather(buf_th, [row_ids, bth + (j + 1)])
                d1b = d1b + jnp.abs(hv2 + rv2 - tv2)
                d2b = d2b + jnp.abs(hhv2 + rv2 - thv2)
                return (d1a, d2a, d1b, d2b)

            d1a, d2a, d1b, d2b = plsc.parallel_loop(
                0, D, 2, unroll=4, carry=(zero, zero, zero, zero)
            )(cols)
            m = jnp.maximum(MARGIN + (d1a + d1b) - (d2a + d2b), 0.0)
            out_v[pl.ds(si * S + g * L, L)] = m
            return gcarry

        lax.fori_loop(0, S // L, group, 0)

    cps = fire(0, 0)
    for si in range(NSUB):
        for cp in cps:
            cp.wait()
        if si + 1 < NSUB:
            nxt = fire(si + 1, (si + 1) % 2)
        compute(si, si % 2)
        if si + 1 < NSUB:
            cps = nxt

    pltpu.sync_copy(out_v, out_hbm.at[pl.ds(base, BPW)])


@jax.jit
def kernel(heads, relations, tails, h_hat, t_hat, entity_weight, rel_weight):
    ent2 = entity_weight.reshape(entity_weight.shape[0] // 2, PD)
    rel2 = rel_weight.reshape(rel_weight.shape[0] // 2, PD)
    mesh = plsc.VectorSubcoreMesh(core_axis_name="c", subcore_axis_name="s")
    fn = pl.kernel(
        _body,
        out_type=jax.ShapeDtypeStruct((B,), jnp.float32),
        mesh=mesh,
        compiler_params=pltpu.CompilerParams(
            needs_layout_passes=False, use_tc_tiling_on_sc=True
        ),
        scratch_types=[
            pltpu.VMEM((BPW,), jnp.int32),
            pltpu.VMEM((BPW,), jnp.int32),
            pltpu.VMEM((BPW,), jnp.int32),
            pltpu.VMEM((BPW,), jnp.int32),
            pltpu.VMEM((BPW,), jnp.int32),
            pltpu.VMEM((BPW,), jnp.int32),
            pltpu.VMEM((BPW,), jnp.int32),
            pltpu.VMEM((BPW,), jnp.int32),
            pltpu.VMEM((BPW,), jnp.int32),
            pltpu.VMEM((BPW,), jnp.int32),
            pltpu.VMEM((2, 5, S, PD), jnp.float32),
            pltpu.VMEM((BPW,), jnp.float32),
            pltpu.SemaphoreType.DMA,
        ],
    )
    out = fn(heads, relations, tails, h_hat, t_hat, ent2, rel2)
    return out[:, None]
